# baseline (speedup 1.0000x reference)
import functools

import jax
import jax.numpy as jnp
from jax import lax
from jax.experimental import pallas as pl
from jax.experimental.pallas import tpu as pltpu
from jax.experimental.pallas import tpu_sc as plsc

NC = 2
NS = 16
NW = NC * NS
L = 16
EPS = 1e-5
_BN_INV = float(1.0 / (1.0 + EPS) ** 0.5)

D = 128
ROWS = 316
AWORDS = ROWS * D
TRASH = AWORDS
ACC = AWORDS + D
CHUNK = 48
NBUF = 5
CAPW = 11648


def _make_partition(e: int):
  epw = e // NW
  vregs = epw // L
  mesh = plsc.VectorSubcoreMesh(core_axis_name="c", subcore_axis_name="s")

  @functools.partial(
      pl.kernel,
      out_type=(
          jax.ShapeDtypeStruct((NW * CAPW,), jnp.int32),
          jax.ShapeDtypeStruct((NW * CAPW,), jnp.int32),
          jax.ShapeDtypeStruct((NW * 64,), jnp.int32),
      ),
      mesh=mesh,
      compiler_params=pltpu.CompilerParams(needs_layout_passes=False),
      scratch_types=[
          pltpu.VMEM((epw,), jnp.int32),
          pltpu.VMEM((epw,), jnp.int32),
          pltpu.VMEM((CAPW,), jnp.int32),
          pltpu.VMEM((CAPW,), jnp.int32),
          pltpu.VMEM((64,), jnp.int32),
      ],
  )
  def part(src, dst, psrc, pdstw, pmeta, sv, dv, sb, db, meta):
    cid = lax.axis_index("c")
    sid = lax.axis_index("s")
    wid = sid * NC + cid
    base = wid * epw
    pltpu.sync_copy(src.at[pl.ds(base, epw)], sv)
    pltpu.sync_copy(dst.at[pl.ds(base, epw)], dv)

    iota = lax.iota(jnp.int32, L)
    padsrc = (iota * 613 + wid * 59) & 8191
    paddstw = jnp.full((L,), TRASH, jnp.int32)

    off = 0
    mvals = []
    for b in range(32):
      lo = b * ROWS
      hi = lo + ROWS

      def body(i, o, lo=lo, hi=hi):
        s16 = sv[pl.ds(i * L, L)]
        d16 = dv[pl.ds(i * L, L)]
        mask = (d16 >= lo) & (d16 < hi)
        na = plsc.all_reduce_population_count(mask)[0]
        plsc.store_compressed(sb.at[pl.ds(o, L)], s16, mask=mask)
        plsc.store_compressed(db.at[pl.ds(o, L)], (d16 - lo) * D, mask=mask)
        return o + na

      noff = lax.fori_loop(0, vregs, body, off)
      for k in range(CHUNK // L + 1):
        sb[pl.ds(noff + k * L, L)] = padsrc
        db[pl.ds(noff + k * L, L)] = paddstw
      poff = ((noff + CHUNK - 1) // CHUNK) * CHUNK
      mvals.append(off)
      mvals.append((poff - off) // CHUNK)
      off = poff

    for r in range(4):
      v = jnp.zeros((L,), jnp.int32)
      for l in range(L):
        v = jnp.where(iota == l, mvals[r * L + l], v)
      meta[pl.ds(r * L, L)] = v

    pltpu.sync_copy(sb, psrc.at[pl.ds(wid * CAPW, CAPW)])
    pltpu.sync_copy(db, pdstw.at[pl.ds(wid * CAPW, CAPW)])
    pltpu.sync_copy(meta, pmeta.at[pl.ds(wid * 64, 64)])

  return part


def _make_agg(npad: int, e: int):
  mesh = plsc.VectorSubcoreMesh(core_axis_name="c", subcore_axis_name="s")
  group = CHUNK * NBUF

  @functools.partial(
      pl.kernel,
      out_type=jax.ShapeDtypeStruct((npad * D,), jnp.float32),
      mesh=mesh,
      compiler_params=pltpu.CompilerParams(needs_layout_passes=False),
      scratch_types=[
          pltpu.VMEM((ACC,), jnp.float32),
          pltpu.VMEM((group,), jnp.int32),
          pltpu.VMEM((group,), jnp.int32),
          pltpu.VMEM((64,), jnp.int32),
          [pltpu.VMEM((CHUNK, D), jnp.float32) for _ in range(NBUF)],
          [pltpu.SemaphoreType.DMA for _ in range(NBUF)],
      ],
  )
  def agg(table, psrc, pdstw, pmeta_t, zeros, out, acc, srcv, dstv, meta,
          stages, sems):
    cid = lax.axis_index("c")
    sid = lax.axis_index("s")
    t = sid * NC + cid
    pltpu.sync_copy(zeros, acc)
    pltpu.sync_copy(pmeta_t.at[pl.ds(t * 64, 64)], meta)

    cvs = [lax.iota(jnp.int32, L) + 16 * v for v in range(D // L)]
    iota = lax.iota(jnp.int32, L)
    mrows = [meta[pl.ds(r * L, L)] for r in range(4)]

    def meta_at(j):
      acc_v = jnp.zeros((L,), jnp.int32)
      for r in range(4):
        acc_v = acc_v + jnp.where(iota == j - r * L, mrows[r], 0)
      return jnp.sum(acc_v)

    def accum_chunk(buf):
      def qbody(q, carry):
        dwin = dstv[pl.ds(buf * CHUNK + q * L, L)]
        for l in range(L):
          dw = dwin[l]
          for v in range(D // L):
            vals = stages[buf][q * L + l, pl.ds(16 * v, L)]
            plsc.addupdate_scatter(acc, [dw + cvs[v]], vals)
        return carry
      lax.fori_loop(0, CHUNK // L, qbody, 0)

    def wbody(w, carry):
      boff = pl.multiple_of(meta_at(2 * w), 16)
      nch = meta_at(2 * w + 1)
      base = w * CAPW + boff
      nfull = nch // NBUF

      def gbody(g, carry2):
        goff = base + g * group
        pltpu.sync_copy(psrc.at[pl.ds(goff, group)], srcv)
        pltpu.sync_copy(pdstw.at[pl.ds(goff, group)], dstv)
        copies = []
        for b in range(NBUF):
          cp = pltpu.make_async_copy(
              table.at[srcv.at[pl.ds(b * CHUNK, CHUNK)]], stages[b], sems[b])
          cp.start()
          copies.append(cp)
        for b in range(NBUF):
          copies[b].wait()
          accum_chunk(b)
        return carry2

      lax.fori_loop(0, nfull, gbody, 0)

      rem = nch - nfull * NBUF
      eoff = base + nfull * group
      for k in range(NBUF - 1):
        @pl.when(k < rem)
        def _():
          coff = eoff + k * CHUNK
          pltpu.sync_copy(psrc.at[pl.ds(coff, CHUNK)],
                          srcv.at[pl.ds(k * CHUNK, CHUNK)])
          pltpu.sync_copy(pdstw.at[pl.ds(coff, CHUNK)],
                          dstv.at[pl.ds(k * CHUNK, CHUNK)])
          pltpu.async_copy(
              table.at[srcv.at[pl.ds(k * CHUNK, CHUNK)]], stages[k],
              sems[k]).wait()
          accum_chunk(k)
      return carry

    lax.fori_loop(0, NW, wbody, 0)
    pltpu.sync_copy(acc.at[pl.ds(0, AWORDS)],
                    out.at[pl.ds(t * AWORDS, AWORDS)])

  return agg


def _mm1_body(x_ref, w_ref, o_ref):
  n = x_ref.shape[0]
  o_ref[:n, :] = jnp.dot(x_ref[...], w_ref[...],
                         preferred_element_type=jnp.float32)
  o_ref[n:, :] = jnp.zeros((o_ref.shape[0] - n, o_ref.shape[1]), jnp.float32)


def _fuse_body(p_ref, b_ref, g_ref, be_ref, w_ref, o_ref):
  z = (p_ref[...] + b_ref[...]) * (g_ref[...] * _BN_INV) + be_ref[...]
  z = jnp.maximum(z, 0.0)
  o_ref[...] = jnp.dot(z, w_ref[...], preferred_element_type=jnp.float32)


def _final_body(p_ref, b_ref, o_ref):
  n, c = o_ref.shape
  z = p_ref[:n, :c] + b_ref[...]
  m = jnp.max(z, axis=-1, keepdims=True)
  s = jnp.log(jnp.sum(jnp.exp(z - m), axis=-1, keepdims=True))
  o_ref[...] = z - m - s


def _tc(body, out_shape, *args):
  return pl.pallas_call(body, out_shape=out_shape)(*args)


def kernel(x, edge_index, W1, b1, g1, be1, W2, b2, g2, be2, W3, b3):
  n, ddim = x.shape
  e = edge_index.shape[1]
  h = W1.shape[1]
  c = W3.shape[1]
  npad = 32 * ROWS
  cpad = 128

  src = edge_index[0]
  dst = edge_index[1]
  zeros_acc = jnp.zeros((ACC,), jnp.float32)
  w3p = jnp.zeros((h, cpad), jnp.float32).at[:, :c].set(W3)

  b1r, g1r, be1r = b1[None, :], g1[None, :], be1[None, :]
  b2r, g2r, be2r = b2[None, :], g2[None, :], be2[None, :]
  b3r = b3[None, :]

  part = _make_partition(e)
  psrc, pdstw, pmeta = part(src, dst)
  pmeta_t = pmeta.reshape(NW, 32, 2).transpose(1, 0, 2).reshape(32 * NW * 2)

  agg = _make_agg(npad, e)

  f32 = jnp.float32
  h1 = _tc(_mm1_body, jax.ShapeDtypeStruct((npad, h), f32), x, W1)
  p1 = agg(h1, psrc, pdstw, pmeta_t, zeros_acc).reshape(npad, h)
  h2 = _tc(_fuse_body, jax.ShapeDtypeStruct((npad, h), f32),
           p1, b1r, g1r, be1r, W2)
  p2 = agg(h2, psrc, pdstw, pmeta_t, zeros_acc).reshape(npad, h)
  h3 = _tc(_fuse_body, jax.ShapeDtypeStruct((npad, cpad), f32),
           p2, b2r, g2r, be2r, w3p)
  p3 = agg(h3, psrc, pdstw, pmeta_t, zeros_acc).reshape(npad, cpad)
  out = _tc(_final_body, jax.ShapeDtypeStruct((n, c), f32), p3, b3r)
  return out
